# baseline (speedup 1.0000x reference)
import jax
import jax.numpy as jnp
import numpy as np
from jax.experimental import pallas as pl

TOPK = 16
N = 4096
D = 512
H = 128
BLOCK = 256
NLANE = 128
NCH = N // NLANE
KEEP = 4
BIGI = np.int32(2**31 - 1)


def _proj_kernel(x_ref, w_ref, b_ref, h_ref):
    h_ref[...] = (
        jnp.dot(x_ref[...], w_ref[...], preferred_element_type=jnp.float32)
        + b_ref[...]
    )


def _select16(cv, ci, sentinel):
    cols = []
    mj = None
    idx = None
    for _ in range(TOPK):
        mj = jnp.max(cv, axis=1, keepdims=True)
        idx = jnp.min(jnp.where(cv == mj, ci, BIGI), axis=1)
        cols.append(idx)
        cv = jnp.where((cv == mj) & (ci == idx[:, None]), sentinel, cv)
    return jnp.stack(cols, axis=1), mj[:, 0], idx


def _adj_topk_kernel(hblk_ref, hall_ref, adj_ref, idx_ref):
    hb = hblk_ref[...]
    ha = hall_ref[...]
    sim = jax.lax.dot_general(
        hb, ha, (((1,), (1,)), ((), ())), preferred_element_type=jnp.float32
    )
    m = jnp.max(sim, axis=1, keepdims=True)
    e = jnp.exp(sim - m)
    adj = e / jnp.sum(e, axis=1, keepdims=True)
    adj_ref[...] = adj

    iota = jax.lax.broadcasted_iota(jnp.int32, (BLOCK, N), 1)
    laneiota = jax.lax.broadcasted_iota(jnp.int32, (BLOCK, NLANE), 1)

    mv = [jnp.full((BLOCK, NLANE), -1.0, jnp.float32) for _ in range(KEEP)]
    mi = [jnp.full((BLOCK, NLANE), BIGI, jnp.int32) for _ in range(KEEP)]
    for c in range(NCH):
        dv = adj[:, c * NLANE:(c + 1) * NLANE]
        di = laneiota + c * NLANE
        for k in range(KEEP):
            gt = dv > mv[k]
            nmv = jnp.where(gt, dv, mv[k])
            nmi = jnp.where(gt, di, mi[k])
            dv, di = jnp.where(gt, mv[k], dv), jnp.where(gt, mi[k], di)
            mv[k], mi[k] = nmv, nmi

    cv = jnp.concatenate(mv, axis=1)
    ci = jnp.concatenate(mi, axis=1)
    fast, val16, idx16 = _select16(cv, ci, -2.0)

    gt16 = (adj > val16[:, None]) | (
        (adj == val16[:, None]) & (iota < idx16[:, None])
    )
    cnt = jnp.sum(gt16.astype(jnp.int32), axis=1)
    ok = jnp.all(cnt == 15)

    idx_ref[...] = fast

    @pl.when(jnp.logical_not(ok))
    def _fallback():
        full, _, _ = _select16(adj, iota, -1.0)
        idx_ref[...] = full


def kernel(x, W, b):
    h = pl.pallas_call(
        _proj_kernel,
        out_shape=jax.ShapeDtypeStruct((N, H), jnp.float32),
    )(x, W, b.reshape(1, H))

    grid = (N // BLOCK,)
    adj, idx = pl.pallas_call(
        _adj_topk_kernel,
        grid=grid,
        in_specs=[
            pl.BlockSpec((BLOCK, H), lambda i: (i, 0)),
            pl.BlockSpec((N, H), lambda i: (0, 0)),
        ],
        out_specs=[
            pl.BlockSpec((BLOCK, N), lambda i: (i, 0)),
            pl.BlockSpec((BLOCK, TOPK), lambda i: (i, 0)),
        ],
        out_shape=[
            jax.ShapeDtypeStruct((N, N), jnp.float32),
            jax.ShapeDtypeStruct((N, TOPK), jnp.int32),
        ],
    )(h, h)

    src = jnp.repeat(jnp.arange(N, dtype=jnp.int64), TOPK)
    dst = idx.reshape(-1).astype(jnp.int64)
    edge_index = jnp.stack([src, dst], axis=0)
    return adj, edge_index


# speedup vs baseline: 1.5482x; 1.5482x over previous
import jax
import jax.numpy as jnp
from jax.experimental import pallas as pl
from jax.experimental.pallas import tpu as pltpu

TOPK = 16
N = 4096
D = 512
H = 128
BLOCK = 256


def _proj_kernel(x_ref, w_ref, b_ref, h_ref):
    h_ref[...] = (
        jnp.dot(x_ref[...], w_ref[...], preferred_element_type=jnp.float32)
        + b_ref[...]
    )


def _adj_topk_kernel(hblk_ref, hall_ref, adj_ref, idx_ref):
    hb = hblk_ref[...]
    ha = hall_ref[...]
    sim = jax.lax.dot_general(
        hb, ha, (((1,), (1,)), ((), ())), preferred_element_type=jnp.float32
    )
    m = jnp.max(sim, axis=1, keepdims=True)
    e = jnp.exp(sim - m)
    s = jnp.sum(e, axis=1, keepdims=True)
    adj = e / s
    adj_ref[...] = adj

    iota = jax.lax.broadcasted_iota(jnp.int32, (BLOCK, N), 1)
    vals = adj
    cols = []
    for _ in range(TOPK):
        mj = jnp.max(vals, axis=1, keepdims=True)
        idx = jnp.min(jnp.where(vals == mj, iota, N), axis=1)
        cols.append(idx)
        vals = jnp.where(iota == idx[:, None], -1.0, vals)
    idx_ref[...] = jnp.stack(cols, axis=1)


def kernel(x, W, b):
    h = pl.pallas_call(
        _proj_kernel,
        out_shape=jax.ShapeDtypeStruct((N, H), jnp.float32),
    )(x, W, b.reshape(1, H))

    grid = (N // BLOCK,)
    adj, idx = pl.pallas_call(
        _adj_topk_kernel,
        grid=grid,
        in_specs=[
            pl.BlockSpec((BLOCK, H), lambda i: (i, 0)),
            pl.BlockSpec((N, H), lambda i: (0, 0)),
        ],
        out_specs=[
            pl.BlockSpec((BLOCK, N), lambda i: (i, 0)),
            pl.BlockSpec((BLOCK, TOPK), lambda i: (i, 0)),
        ],
        out_shape=[
            jax.ShapeDtypeStruct((N, N), jnp.float32),
            jax.ShapeDtypeStruct((N, TOPK), jnp.int32),
        ],
    )(h, h)

    src = jnp.repeat(jnp.arange(N, dtype=jnp.int64), TOPK)
    dst = idx.reshape(-1).astype(jnp.int64)
    edge_index = jnp.stack([src, dst], axis=0)
    return adj, edge_index
